# Initial kernel scaffold; baseline (speedup 1.0000x reference)
import functools

import jax
import jax.numpy as jnp
from jax import lax
from jax.experimental import pallas as pl
from jax.experimental.pallas import tpu as pltpu
from jax.experimental.pallas import tpu_sc as plsc

N = 10000
E = 320000
NC = 2
NS = 16
NW = NC * NS
K = 128
CPW = 80
EP = NW * CPW * K
RPS = N // NS

_f32 = jnp.float32


def _mesh():
    return plsc.VectorSubcoreMesh(core_axis_name="c", subcore_axis_name="s")


def _deg_body(ew8_h, dst_h, zero_h, out_h, dstv, rows, acc_sh):
    c = lax.axis_index("c")
    s = lax.axis_index("s")

    @pl.when(s == 0)
    def _():
        pltpu.sync_copy(zero_h, acc_sh)
    plsc.subcore_barrier()

    w = s * NC + c
    base = w * (CPW * K)

    def chunk(i, carry):
        off = base + i * K
        pltpu.sync_copy(dst_h.at[pl.ds(off, K)], dstv)
        pltpu.sync_copy(ew8_h.at[pl.ds(off, K)], rows)
        pltpu.sync_copy(rows, acc_sh.at[dstv], add=True)
        return carry

    lax.fori_loop(0, CPW, chunk, 0)
    plsc.subcore_barrier()

    @pl.when(s == 0)
    def _():
        pltpu.sync_copy(acc_sh, out_h.at[pl.ds(c * N, N)])


_deg_pass = pl.kernel(
    _deg_body,
    out_type=jax.ShapeDtypeStruct((NC * N, 8), _f32),
    mesh=_mesh(),
    scratch_types=[
        pltpu.VMEM((K,), jnp.int32),
        pltpu.VMEM((K, 8), _f32),
        pltpu.VMEM_SHARED((N, 8), _f32),
    ],
)


def _msg_body(D, table_h, src_h, dst_h, ew_h, zero_h, out_h,
              srcv, dstv, ewv, rows, acc_sh):
    c = lax.axis_index("c")
    s = lax.axis_index("s")
    row0 = s * RPS

    pltpu.sync_copy(zero_h.at[pl.ds(row0, RPS)], acc_sh.at[pl.ds(row0, RPS)])
    plsc.subcore_barrier()

    w = s * NC + c
    base = w * (CPW * K)
    nvec = D // 16

    def chunk(i, carry):
        off = base + i * K
        pltpu.sync_copy(src_h.at[pl.ds(off, K)], srcv)
        pltpu.sync_copy(dst_h.at[pl.ds(off, K)], dstv)
        pltpu.sync_copy(ew_h.at[pl.ds(off, K)], ewv)
        pltpu.sync_copy(table_h.at[srcv], rows)

        def grp(g, carry2):
            e0 = g * 16
            for l in range(16):
                e = e0 + l
                bc = plsc.load_gather(ewv, [jnp.full((16,), e, jnp.int32)])
                for j in range(nvec):
                    rows[e, pl.ds(j * 16, 16)] = rows[e, pl.ds(j * 16, 16)] * bc
            return carry2

        lax.fori_loop(0, K // 16, grp, 0)
        pltpu.sync_copy(rows, acc_sh.at[dstv], add=True)
        return carry

    lax.fori_loop(0, CPW, chunk, 0)
    plsc.subcore_barrier()
    pltpu.sync_copy(acc_sh.at[pl.ds(row0, RPS)],
                    out_h.at[pl.ds(c * N + row0, RPS)])


def _make_msg_pass(D):
    return pl.kernel(
        functools.partial(_msg_body, D),
        out_type=jax.ShapeDtypeStruct((NC * N, D), _f32),
        mesh=_mesh(),
        scratch_types=[
            pltpu.VMEM((K,), jnp.int32),
            pltpu.VMEM((K,), jnp.int32),
            pltpu.VMEM((K,), _f32),
            pltpu.VMEM((K, D), _f32),
            pltpu.VMEM_SHARED((N, D), _f32),
        ],
    )


_msg128 = _make_msg_pass(128)
_msg64 = _make_msg_pass(64)


_BLK = 1000
_GRID = N // _BLK


def _tcb_body(x_ref, w_ref, degp_ref, hs_ref, dinv_ref):
    deg = degp_ref[0, :, 0:1] + degp_ref[1, :, 0:1] + 1.0
    dinv = lax.rsqrt(deg)
    dinv_ref[...] = dinv
    hs_ref[...] = jnp.dot(x_ref[...], w_ref[...],
                          preferred_element_type=_f32) * dinv


def _tc_scale_matmul(x, W1, degp):
    return pl.pallas_call(
        _tcb_body,
        grid=(_GRID,),
        in_specs=[
            pl.BlockSpec((_BLK, 128), lambda i: (i, 0)),
            pl.BlockSpec((128, 128), lambda i: (0, 0)),
            pl.BlockSpec((2, _BLK, 8), lambda i: (0, i, 0)),
        ],
        out_specs=[
            pl.BlockSpec((_BLK, 128), lambda i: (i, 0)),
            pl.BlockSpec((_BLK, 1), lambda i: (i, 0)),
        ],
        out_shape=[
            jax.ShapeDtypeStruct((N, 128), _f32),
            jax.ShapeDtypeStruct((N, 1), _f32),
        ],
    )(x, W1, degp)


def _tcc_body(acc_ref, hs1_ref, dinv_ref, b1_ref, w2_ref, xemb_ref, hs2_ref):
    dinv = dinv_ref[...]
    t = (acc_ref[0] + acc_ref[1] + hs1_ref[...]) * dinv + b1_ref[...]
    xemb_ref[...] = t
    h = jnp.maximum(t, 0.0)
    hs2_ref[...] = jnp.dot(h, w2_ref[...], preferred_element_type=_f32) * dinv


def _tc_mid(acc1, hs1, dinv, b1, W2):
    return pl.pallas_call(
        _tcc_body,
        grid=(_GRID,),
        in_specs=[
            pl.BlockSpec((2, _BLK, 128), lambda i: (0, i, 0)),
            pl.BlockSpec((_BLK, 128), lambda i: (i, 0)),
            pl.BlockSpec((_BLK, 1), lambda i: (i, 0)),
            pl.BlockSpec((1, 128), lambda i: (0, 0)),
            pl.BlockSpec((128, 64), lambda i: (0, 0)),
        ],
        out_specs=[
            pl.BlockSpec((_BLK, 128), lambda i: (i, 0)),
            pl.BlockSpec((_BLK, 64), lambda i: (i, 0)),
        ],
        out_shape=[
            jax.ShapeDtypeStruct((N, 128), _f32),
            jax.ShapeDtypeStruct((N, 64), _f32),
        ],
    )(acc1, hs1, dinv, b1, W2)


def _tcd_body(acc_ref, hs2_ref, dinv_ref, b2_ref, o_ref):
    o_ref[...] = ((acc_ref[0] + acc_ref[1] + hs2_ref[...]) * dinv_ref[...]
                  + b2_ref[...])


def _tc_out(acc2, hs2, dinv, b2):
    return pl.pallas_call(
        _tcd_body,
        grid=(_GRID,),
        in_specs=[
            pl.BlockSpec((2, _BLK, 64), lambda i: (0, i, 0)),
            pl.BlockSpec((_BLK, 64), lambda i: (i, 0)),
            pl.BlockSpec((_BLK, 1), lambda i: (i, 0)),
            pl.BlockSpec((1, 64), lambda i: (0, 0)),
        ],
        out_specs=pl.BlockSpec((_BLK, 64), lambda i: (i, 0)),
        out_shape=jax.ShapeDtypeStruct((N, 64), _f32),
    )(acc2, hs2, dinv, b2)


def kernel(x, edge_index, edge_attr, W1, b1, W2, b2):
    pad = EP - E
    src = jnp.concatenate([edge_index[0], jnp.zeros((pad,), jnp.int32)])
    dst = jnp.concatenate([edge_index[1], jnp.zeros((pad,), jnp.int32)])
    ew = jnp.concatenate([edge_attr, jnp.zeros((pad,), _f32)])
    ew8 = jnp.broadcast_to(ew[:, None], (EP, 8))

    zero8 = jnp.zeros((N, 8), _f32)
    zero128 = jnp.zeros((N, 128), _f32)
    zero64 = jnp.zeros((N, 64), _f32)

    degp = _deg_pass(ew8, dst, zero8).reshape(NC, N, 8)
    hs1, dinv = _tc_scale_matmul(x, W1, degp)
    acc1 = _msg128(hs1, src, dst, ew, zero128).reshape(NC, N, 128)
    x_emb, hs2 = _tc_mid(acc1, hs1, dinv, b1.reshape(1, 128), W2)
    acc2 = _msg64(hs2, src, dst, ew, zero64).reshape(NC, N, 64)
    out = _tc_out(acc2, hs2, dinv, b2.reshape(1, 64))
    return (out, x_emb)


# speedup vs baseline: 5.9806x; 5.9806x over previous
import functools

import jax
import jax.numpy as jnp
from jax import lax
from jax.experimental import pallas as pl
from jax.experimental.pallas import tpu as pltpu
from jax.experimental.pallas import tpu_sc as plsc

N = 10000
NP_ = 10240
E = 320000
NC = 2
NS = 16
NW = NC * NS
K = 128
CPW = 80
EP = NW * CPW * K
RPS = NP_ // NS

_f32 = jnp.float32


def _mesh():
    return plsc.VectorSubcoreMesh(core_axis_name="c", subcore_axis_name="s")


def _deg_body(ew8_h, dst_h, zero_h, out_h, dstv, rows, acc_sh):
    c = lax.axis_index("c")
    s = lax.axis_index("s")

    @pl.when(s == 0)
    def _():
        pltpu.sync_copy(zero_h, acc_sh)
    plsc.subcore_barrier()

    w = s * NC + c
    base = w * (CPW * K)

    def chunk(i, carry):
        off = base + i * K
        pltpu.sync_copy(dst_h.at[pl.ds(off, K)], dstv)
        pltpu.sync_copy(ew8_h.at[pl.ds(off, K)], rows)
        pltpu.sync_copy(rows, acc_sh.at[dstv], add=True)
        return carry

    lax.fori_loop(0, CPW, chunk, 0)
    plsc.subcore_barrier()

    @pl.when(s == 0)
    def _():
        pltpu.sync_copy(acc_sh, out_h.at[pl.ds(c * NP_, NP_)])


_deg_pass = pl.kernel(
    _deg_body,
    out_type=jax.ShapeDtypeStruct((NC * NP_, 8), _f32),
    mesh=_mesh(),
    compiler_params=pltpu.CompilerParams(needs_layout_passes=False, use_tc_tiling_on_sc=False),
    scratch_types=[
        pltpu.VMEM((K,), jnp.int32),
        pltpu.VMEM((K, 8), _f32),
        pltpu.VMEM_SHARED((NP_, 8), _f32),
    ],
)


def _msg_body(D, table_h, src_h, dst_h, ew_h, zero_h, out_h,
              srcv, dstv, ewv, rows, acc_sh):
    c = lax.axis_index("c")
    s = lax.axis_index("s")
    row0 = s * RPS

    pltpu.sync_copy(zero_h.at[pl.ds(row0, RPS)], acc_sh.at[pl.ds(row0, RPS)])
    plsc.subcore_barrier()

    w = s * NC + c
    base = w * (CPW * K)
    nvec = D // 16

    def chunk(i, carry):
        off = base + i * K
        pltpu.sync_copy(src_h.at[pl.ds(off, K)], srcv)
        pltpu.sync_copy(dst_h.at[pl.ds(off, K)], dstv)
        pltpu.sync_copy(ew_h.at[pl.ds(off, K)], ewv)
        pltpu.sync_copy(table_h.at[srcv], rows)

        def grp(g, carry2):
            e0 = g * 16
            for l in range(16):
                e = e0 + l
                bc = plsc.load_gather(ewv, [jnp.full((16,), e, jnp.int32)])
                for j in range(nvec):
                    rows[e, pl.ds(j * 16, 16)] = rows[e, pl.ds(j * 16, 16)] * bc
            return carry2

        lax.fori_loop(0, K // 16, grp, 0)
        pltpu.sync_copy(rows, acc_sh.at[dstv], add=True)
        return carry

    lax.fori_loop(0, CPW, chunk, 0)
    plsc.subcore_barrier()
    pltpu.sync_copy(acc_sh.at[pl.ds(row0, RPS)],
                    out_h.at[pl.ds(c * NP_ + row0, RPS)])


def _make_msg_pass(D):
    return pl.kernel(
        functools.partial(_msg_body, D),
        out_type=jax.ShapeDtypeStruct((NC * NP_, D), _f32),
        mesh=_mesh(),
        compiler_params=pltpu.CompilerParams(needs_layout_passes=False, use_tc_tiling_on_sc=False),
        scratch_types=[
            pltpu.VMEM((K,), jnp.int32),
            pltpu.VMEM((K,), jnp.int32),
            pltpu.VMEM((K,), _f32),
            pltpu.VMEM((K, D), _f32),
            pltpu.VMEM_SHARED((NP_, D), _f32),
        ],
    )


_msg128 = _make_msg_pass(128)
_msg64 = _make_msg_pass(64)


_BLK = 1024
_GRID = NP_ // _BLK


def _tcb_body(x_ref, w_ref, degp_ref, hs_ref, dinv_ref):
    deg = degp_ref[0, :, 0:1] + degp_ref[1, :, 0:1] + 1.0
    dinv = lax.rsqrt(deg)
    dinv_ref[...] = dinv
    hs_ref[...] = jnp.dot(x_ref[...], w_ref[...],
                          preferred_element_type=_f32) * dinv


def _tc_scale_matmul(x, W1, degp):
    return pl.pallas_call(
        _tcb_body,
        grid=(_GRID,),
        in_specs=[
            pl.BlockSpec((_BLK, 128), lambda i: (i, 0)),
            pl.BlockSpec((128, 128), lambda i: (0, 0)),
            pl.BlockSpec((2, _BLK, 8), lambda i: (0, i, 0)),
        ],
        out_specs=[
            pl.BlockSpec((_BLK, 128), lambda i: (i, 0)),
            pl.BlockSpec((_BLK, 1), lambda i: (i, 0)),
        ],
        out_shape=[
            jax.ShapeDtypeStruct((NP_, 128), _f32),
            jax.ShapeDtypeStruct((NP_, 1), _f32),
        ],
    )(x, W1, degp)


def _tcc_body(acc_ref, hs1_ref, dinv_ref, b1_ref, w2_ref, xemb_ref, hs2_ref):
    dinv = dinv_ref[...]
    t = (acc_ref[0] + acc_ref[1] + hs1_ref[...]) * dinv + b1_ref[...]
    xemb_ref[...] = t
    h = jnp.maximum(t, 0.0)
    hs2_ref[...] = jnp.dot(h, w2_ref[...], preferred_element_type=_f32) * dinv


def _tc_mid(acc1, hs1, dinv, b1, W2):
    return pl.pallas_call(
        _tcc_body,
        grid=(_GRID,),
        in_specs=[
            pl.BlockSpec((2, _BLK, 128), lambda i: (0, i, 0)),
            pl.BlockSpec((_BLK, 128), lambda i: (i, 0)),
            pl.BlockSpec((_BLK, 1), lambda i: (i, 0)),
            pl.BlockSpec((1, 128), lambda i: (0, 0)),
            pl.BlockSpec((128, 64), lambda i: (0, 0)),
        ],
        out_specs=[
            pl.BlockSpec((_BLK, 128), lambda i: (i, 0)),
            pl.BlockSpec((_BLK, 64), lambda i: (i, 0)),
        ],
        out_shape=[
            jax.ShapeDtypeStruct((NP_, 128), _f32),
            jax.ShapeDtypeStruct((NP_, 64), _f32),
        ],
    )(acc1, hs1, dinv, b1, W2)


def _tcd_body(acc_ref, hs2_ref, dinv_ref, b2_ref, o_ref):
    o_ref[...] = ((acc_ref[0] + acc_ref[1] + hs2_ref[...]) * dinv_ref[...]
                  + b2_ref[...])


def _tc_out(acc2, hs2, dinv, b2):
    return pl.pallas_call(
        _tcd_body,
        grid=(_GRID,),
        in_specs=[
            pl.BlockSpec((2, _BLK, 64), lambda i: (0, i, 0)),
            pl.BlockSpec((_BLK, 64), lambda i: (i, 0)),
            pl.BlockSpec((_BLK, 1), lambda i: (i, 0)),
            pl.BlockSpec((1, 64), lambda i: (0, 0)),
        ],
        out_specs=pl.BlockSpec((_BLK, 64), lambda i: (i, 0)),
        out_shape=jax.ShapeDtypeStruct((NP_, 64), _f32),
    )(acc2, hs2, dinv, b2)


def kernel(x, edge_index, edge_attr, W1, b1, W2, b2):
    pad = EP - E
    src = jnp.concatenate([edge_index[0], jnp.zeros((pad,), jnp.int32)])
    dst = jnp.concatenate([edge_index[1], jnp.zeros((pad,), jnp.int32)])
    ew = jnp.concatenate([edge_attr, jnp.zeros((pad,), _f32)])
    ew8 = jnp.broadcast_to(ew[:, None], (EP, 8))

    xp = jnp.concatenate([x, jnp.zeros((NP_ - N, 128), _f32)])

    zero8 = jnp.zeros((NP_, 8), _f32)
    zero128 = jnp.zeros((NP_, 128), _f32)
    zero64 = jnp.zeros((NP_, 64), _f32)

    degp = _deg_pass(ew8, dst, zero8).reshape(NC, NP_, 8)
    hs1, dinv = _tc_scale_matmul(xp, W1, degp)
    acc1 = _msg128(hs1, src, dst, ew, zero128).reshape(NC, NP_, 128)
    x_emb, hs2 = _tc_mid(acc1, hs1, dinv, b1.reshape(1, 128), W2)
    acc2 = _msg64(hs2, src, dst, ew, zero64).reshape(NC, NP_, 64)
    out = _tc_out(acc2, hs2, dinv, b2.reshape(1, 64))
    return (out[:N], x_emb[:N])


# speedup vs baseline: 6.9595x; 1.1637x over previous
import functools

import jax
import jax.numpy as jnp
from jax import lax
from jax.experimental import pallas as pl
from jax.experimental.pallas import tpu as pltpu
from jax.experimental.pallas import tpu_sc as plsc

N = 10000
NP_ = 10240
E = 320000
NC = 2
NS = 16
NW = NC * NS
K = 128
CPW = 80
EP = NW * CPW * K
RPS = NP_ // NS

_f32 = jnp.float32


def _mesh():
    return plsc.VectorSubcoreMesh(core_axis_name="c", subcore_axis_name="s")


DEG_SC = 8


def _deg_body(ew8_h, dst2_h, zero_h, out_h, dstv, rows, sem_s, acc_sh):
    c = lax.axis_index("c")
    s = lax.axis_index("s")

    @pl.when(s == 0)
    def _():
        pltpu.sync_copy(zero_h, acc_sh)
    plsc.subcore_barrier()

    w = s * NC + c
    base_r = w * CPW

    def chunk(g, carry):
        offr = base_r + g * DEG_SC
        pltpu.sync_copy(dst2_h.at[pl.ds(offr, DEG_SC)], dstv)
        pltpu.sync_copy(ew8_h.at[pl.ds(offr * K, DEG_SC * K)], rows)
        descs = [pltpu.async_copy(rows.at[pl.ds(q * K, K)],
                                  acc_sh.at[dstv.at[q]], sem_s, add=True)
                 for q in range(DEG_SC)]
        for d_ in descs:
            d_.wait()
        return carry

    lax.fori_loop(0, CPW // DEG_SC, chunk, 0)
    plsc.subcore_barrier()

    @pl.when(s == 0)
    def _():
        pltpu.sync_copy(acc_sh, out_h.at[pl.ds(c * NP_, NP_)])


_deg_pass = pl.kernel(
    _deg_body,
    out_type=jax.ShapeDtypeStruct((NC * NP_, 8), _f32),
    mesh=_mesh(),
    compiler_params=pltpu.CompilerParams(needs_layout_passes=False, use_tc_tiling_on_sc=False),
    scratch_types=[
        pltpu.VMEM((DEG_SC, K), jnp.int32),
        pltpu.VMEM((DEG_SC * K, 8), _f32),
        pltpu.SemaphoreType.DMA,
        pltpu.VMEM_SHARED((NP_, 8), _f32),
    ],
)


def _msg_body(D, MSG_SC, table_h, src2_h, dst2_h, ew_h, zero_h, out_h,
              srcv, dstv, ewv, rows, sem_g, sem_s, acc_sh):
    c = lax.axis_index("c")
    s = lax.axis_index("s")
    row0 = s * RPS

    pltpu.sync_copy(zero_h.at[pl.ds(row0, RPS)], acc_sh.at[pl.ds(row0, RPS)])
    plsc.subcore_barrier()

    w = s * NC + c
    base_r = w * CPW
    nvec = D // 16

    def chunk(g, carry):
        offr = base_r + g * MSG_SC
        pltpu.sync_copy(src2_h.at[pl.ds(offr, MSG_SC)], srcv)
        pltpu.sync_copy(dst2_h.at[pl.ds(offr, MSG_SC)], dstv)
        pltpu.sync_copy(ew_h.at[pl.ds(offr * K, MSG_SC * K)], ewv)
        gds = [pltpu.async_copy(table_h.at[srcv.at[q]],
                                rows.at[pl.ds(q * K, K)], sem_g)
               for q in range(MSG_SC)]
        for d_ in gds:
            d_.wait()

        def grp(t, carry2):
            e0 = t * 16
            for l in range(16):
                e = e0 + l
                bc = plsc.load_gather(ewv, [jnp.full((16,), e, jnp.int32)])
                for j in range(nvec):
                    rows[e, pl.ds(j * 16, 16)] = rows[e, pl.ds(j * 16, 16)] * bc
            return carry2

        lax.fori_loop(0, (MSG_SC * K) // 16, grp, 0)
        sds = [pltpu.async_copy(rows.at[pl.ds(q * K, K)],
                                acc_sh.at[dstv.at[q]], sem_s, add=True)
               for q in range(MSG_SC)]
        for d_ in sds:
            d_.wait()
        return carry

    lax.fori_loop(0, CPW // MSG_SC, chunk, 0)
    plsc.subcore_barrier()
    pltpu.sync_copy(acc_sh.at[pl.ds(row0, RPS)],
                    out_h.at[pl.ds(c * NP_ + row0, RPS)])


def _make_msg_pass(D, MSG_SC):
    return pl.kernel(
        functools.partial(_msg_body, D, MSG_SC),
        out_type=jax.ShapeDtypeStruct((NC * NP_, D), _f32),
        mesh=_mesh(),
        compiler_params=pltpu.CompilerParams(needs_layout_passes=False, use_tc_tiling_on_sc=False),
        scratch_types=[
            pltpu.VMEM((MSG_SC, K), jnp.int32),
            pltpu.VMEM((MSG_SC, K), jnp.int32),
            pltpu.VMEM((MSG_SC * K,), _f32),
            pltpu.VMEM((MSG_SC * K, D), _f32),
            pltpu.SemaphoreType.DMA,
            pltpu.SemaphoreType.DMA,
            pltpu.VMEM_SHARED((NP_, D), _f32),
        ],
    )


_msg128 = _make_msg_pass(128, 2)
_msg64 = _make_msg_pass(64, 4)


_BLK = 1024
_GRID = NP_ // _BLK


def _tcb_body(x_ref, w_ref, degp_ref, hs_ref, dinv_ref):
    deg = degp_ref[0, :, 0:1] + degp_ref[1, :, 0:1] + 1.0
    dinv = lax.rsqrt(deg)
    dinv_ref[...] = dinv
    hs_ref[...] = jnp.dot(x_ref[...], w_ref[...],
                          preferred_element_type=_f32) * dinv


def _tc_scale_matmul(x, W1, degp):
    return pl.pallas_call(
        _tcb_body,
        grid=(_GRID,),
        in_specs=[
            pl.BlockSpec((_BLK, 128), lambda i: (i, 0)),
            pl.BlockSpec((128, 128), lambda i: (0, 0)),
            pl.BlockSpec((2, _BLK, 8), lambda i: (0, i, 0)),
        ],
        out_specs=[
            pl.BlockSpec((_BLK, 128), lambda i: (i, 0)),
            pl.BlockSpec((_BLK, 1), lambda i: (i, 0)),
        ],
        out_shape=[
            jax.ShapeDtypeStruct((NP_, 128), _f32),
            jax.ShapeDtypeStruct((NP_, 1), _f32),
        ],
    )(x, W1, degp)


def _tcc_body(acc_ref, hs1_ref, dinv_ref, b1_ref, w2_ref, xemb_ref, hs2_ref):
    dinv = dinv_ref[...]
    t = (acc_ref[0] + acc_ref[1] + hs1_ref[...]) * dinv + b1_ref[...]
    xemb_ref[...] = t
    h = jnp.maximum(t, 0.0)
    hs2_ref[...] = jnp.dot(h, w2_ref[...], preferred_element_type=_f32) * dinv


def _tc_mid(acc1, hs1, dinv, b1, W2):
    return pl.pallas_call(
        _tcc_body,
        grid=(_GRID,),
        in_specs=[
            pl.BlockSpec((2, _BLK, 128), lambda i: (0, i, 0)),
            pl.BlockSpec((_BLK, 128), lambda i: (i, 0)),
            pl.BlockSpec((_BLK, 1), lambda i: (i, 0)),
            pl.BlockSpec((1, 128), lambda i: (0, 0)),
            pl.BlockSpec((128, 64), lambda i: (0, 0)),
        ],
        out_specs=[
            pl.BlockSpec((_BLK, 128), lambda i: (i, 0)),
            pl.BlockSpec((_BLK, 64), lambda i: (i, 0)),
        ],
        out_shape=[
            jax.ShapeDtypeStruct((NP_, 128), _f32),
            jax.ShapeDtypeStruct((NP_, 64), _f32),
        ],
    )(acc1, hs1, dinv, b1, W2)


def _tcd_body(acc_ref, hs2_ref, dinv_ref, b2_ref, o_ref):
    o_ref[...] = ((acc_ref[0] + acc_ref[1] + hs2_ref[...]) * dinv_ref[...]
                  + b2_ref[...])


def _tc_out(acc2, hs2, dinv, b2):
    return pl.pallas_call(
        _tcd_body,
        grid=(_GRID,),
        in_specs=[
            pl.BlockSpec((2, _BLK, 64), lambda i: (0, i, 0)),
            pl.BlockSpec((_BLK, 64), lambda i: (i, 0)),
            pl.BlockSpec((_BLK, 1), lambda i: (i, 0)),
            pl.BlockSpec((1, 64), lambda i: (0, 0)),
        ],
        out_specs=pl.BlockSpec((_BLK, 64), lambda i: (i, 0)),
        out_shape=jax.ShapeDtypeStruct((NP_, 64), _f32),
    )(acc2, hs2, dinv, b2)


def kernel(x, edge_index, edge_attr, W1, b1, W2, b2):
    pad = EP - E
    src = jnp.concatenate([edge_index[0], jnp.zeros((pad,), jnp.int32)])
    dst = jnp.concatenate([edge_index[1], jnp.zeros((pad,), jnp.int32)])
    ew = jnp.concatenate([edge_attr, jnp.zeros((pad,), _f32)])
    ew8 = jnp.broadcast_to(ew[:, None], (EP, 8))

    xp = jnp.concatenate([x, jnp.zeros((NP_ - N, 128), _f32)])

    zero8 = jnp.zeros((NP_, 8), _f32)
    zero128 = jnp.zeros((NP_, 128), _f32)
    zero64 = jnp.zeros((NP_, 64), _f32)

    src2 = src.reshape(EP // K, K)
    dst2 = dst.reshape(EP // K, K)

    degp = _deg_pass(ew8, dst2, zero8).reshape(NC, NP_, 8)
    hs1, dinv = _tc_scale_matmul(xp, W1, degp)
    acc1 = _msg128(hs1, src2, dst2, ew, zero128).reshape(NC, NP_, 128)
    x_emb, hs2 = _tc_mid(acc1, hs1, dinv, b1.reshape(1, 128), W2)
    acc2 = _msg64(hs2, src2, dst2, ew, zero64).reshape(NC, NP_, 64)
    out = _tc_out(acc2, hs2, dinv, b2.reshape(1, 64))
    return (out[:N], x_emb[:N])


# speedup vs baseline: 8.6190x; 1.2385x over previous
import functools

import jax
import jax.numpy as jnp
from jax import lax
from jax.experimental import pallas as pl
from jax.experimental.pallas import tpu as pltpu
from jax.experimental.pallas import tpu_sc as plsc

N = 10000
NP_ = 10240
E = 320000
NC = 2
NS = 16
NW = NC * NS
K = 128
CPW = 80
EP = NW * CPW * K
RPS = NP_ // NS

_f32 = jnp.float32


def _mesh():
    return plsc.VectorSubcoreMesh(core_axis_name="c", subcore_axis_name="s")


DEG_SC = 8


def _deg_body(ew_h, dst2_h, zero_h, out_h, dstv, ewv, rows, sem_s, acc_sh):
    c = lax.axis_index("c")
    s = lax.axis_index("s")

    @pl.when(s == 0)
    def _():
        pltpu.sync_copy(zero_h, acc_sh)
    pltpu.sync_copy(zero_h.at[pl.ds(0, DEG_SC * K)], rows)
    plsc.subcore_barrier()

    w = s * NC + c
    base_r = w * CPW
    col0 = jnp.zeros((16,), jnp.int32)
    lane = lax.iota(jnp.int32, 16)

    def chunk(g, carry):
        offr = base_r + g * DEG_SC
        pltpu.sync_copy(dst2_h.at[pl.ds(offr, DEG_SC)], dstv)
        pltpu.sync_copy(ew_h.at[pl.ds(offr * K, DEG_SC * K)], ewv)

        def sg(t, carry2):
            e0 = t * 16
            plsc.store_scatter(rows, [e0 + lane, col0], ewv[pl.ds(e0, 16)])
            return carry2

        lax.fori_loop(0, (DEG_SC * K) // 16, sg, 0)
        descs = [pltpu.async_copy(rows.at[pl.ds(q * K, K)],
                                  acc_sh.at[dstv.at[q]], sem_s, add=True)
                 for q in range(DEG_SC)]
        for d_ in descs:
            d_.wait()
        return carry

    lax.fori_loop(0, CPW // DEG_SC, chunk, 0)
    plsc.subcore_barrier()

    @pl.when(s == 0)
    def _():
        pltpu.sync_copy(acc_sh, out_h.at[pl.ds(c * NP_, NP_)])


_deg_pass = pl.kernel(
    _deg_body,
    out_type=jax.ShapeDtypeStruct((NC * NP_, 8), _f32),
    mesh=_mesh(),
    compiler_params=pltpu.CompilerParams(needs_layout_passes=False, use_tc_tiling_on_sc=False),
    scratch_types=[
        pltpu.VMEM((DEG_SC, K), jnp.int32),
        pltpu.VMEM((DEG_SC * K,), _f32),
        pltpu.VMEM((DEG_SC * K, 8), _f32),
        pltpu.SemaphoreType.DMA,
        pltpu.VMEM_SHARED((NP_, 8), _f32),
    ],
)


def _msg_body(D, MSG_SC, table_h, src2_h, dst2_h, ew_h, out_h,
              srcv, dstv, ewv, rows, sem_g, sem_s, acc_sh):
    c = lax.axis_index("c")
    s = lax.axis_index("s")
    row0 = s * RPS
    nv = D // 16
    z16 = jnp.zeros((16,), _f32)
    NR = MSG_SC * K

    def zr(r, carry):
        for j in range(nv):
            rows[r, pl.ds(j * 16, 16)] = z16
        return carry

    lax.fori_loop(0, NR, zr, 0)
    nfull, rem = RPS // NR, RPS % NR
    for t in range(nfull):
        pltpu.sync_copy(rows, acc_sh.at[pl.ds(row0 + t * NR, NR)])
    if rem:
        pltpu.sync_copy(rows.at[pl.ds(0, rem)],
                        acc_sh.at[pl.ds(row0 + nfull * NR, rem)])
    plsc.subcore_barrier()

    w = s * NC + c
    base_r = w * CPW
    nvec = D // 16

    def chunk(g, carry):
        offr = base_r + g * MSG_SC
        pltpu.sync_copy(src2_h.at[pl.ds(offr, MSG_SC)], srcv)
        pltpu.sync_copy(dst2_h.at[pl.ds(offr, MSG_SC)], dstv)
        pltpu.sync_copy(ew_h.at[pl.ds(offr * K, MSG_SC * K)], ewv)
        gds = [pltpu.async_copy(table_h.at[srcv.at[q]],
                                rows.at[pl.ds(q * K, K)], sem_g)
               for q in range(MSG_SC)]
        for d_ in gds:
            d_.wait()

        def grp(t, carry2):
            e0 = t * 16
            for l in range(16):
                e = e0 + l
                bc = plsc.load_gather(ewv, [jnp.full((16,), e, jnp.int32)])
                for j in range(nvec):
                    rows[e, pl.ds(j * 16, 16)] = rows[e, pl.ds(j * 16, 16)] * bc
            return carry2

        lax.fori_loop(0, (MSG_SC * K) // 16, grp, 0)
        sds = [pltpu.async_copy(rows.at[pl.ds(q * K, K)],
                                acc_sh.at[dstv.at[q]], sem_s, add=True)
               for q in range(MSG_SC)]
        for d_ in sds:
            d_.wait()
        return carry

    lax.fori_loop(0, CPW // MSG_SC, chunk, 0)
    plsc.subcore_barrier()
    pltpu.sync_copy(acc_sh.at[pl.ds(row0, RPS)],
                    out_h.at[pl.ds(c * NP_ + row0, RPS)])


def _make_msg_pass(D, MSG_SC):
    return pl.kernel(
        functools.partial(_msg_body, D, MSG_SC),
        out_type=jax.ShapeDtypeStruct((NC * NP_, D), _f32),
        mesh=_mesh(),
        compiler_params=pltpu.CompilerParams(needs_layout_passes=False, use_tc_tiling_on_sc=False),
        scratch_types=[
            pltpu.VMEM((MSG_SC, K), jnp.int32),
            pltpu.VMEM((MSG_SC, K), jnp.int32),
            pltpu.VMEM((MSG_SC * K,), _f32),
            pltpu.VMEM((MSG_SC * K, D), _f32),
            pltpu.SemaphoreType.DMA,
            pltpu.SemaphoreType.DMA,
            pltpu.VMEM_SHARED((NP_, D), _f32),
        ],
    )


_msg128 = _make_msg_pass(128, 2)
_msg64 = _make_msg_pass(64, 4)


_BLK = 1000
_GRID = N // _BLK


def _tcb_body(x_ref, w_ref, degp_ref, hs_ref, dinv_ref):
    deg = degp_ref[0, :, 0:1] + degp_ref[1, :, 0:1] + 1.0
    dinv = lax.rsqrt(deg)
    dinv_ref[...] = dinv
    hs_ref[...] = jnp.dot(x_ref[...], w_ref[...],
                          preferred_element_type=_f32) * dinv


def _tc_scale_matmul(x, W1, degp):
    return pl.pallas_call(
        _tcb_body,
        grid=(_GRID,),
        in_specs=[
            pl.BlockSpec((_BLK, 128), lambda i: (i, 0)),
            pl.BlockSpec((128, 128), lambda i: (0, 0)),
            pl.BlockSpec((2, _BLK, 8), lambda i: (0, i, 0)),
        ],
        out_specs=[
            pl.BlockSpec((_BLK, 128), lambda i: (i, 0)),
            pl.BlockSpec((_BLK, 1), lambda i: (i, 0)),
        ],
        out_shape=[
            jax.ShapeDtypeStruct((N, 128), _f32),
            jax.ShapeDtypeStruct((N, 1), _f32),
        ],
    )(x, W1, degp)


def _tcc_body(acc_ref, hs1_ref, dinv_ref, b1_ref, w2_ref, xemb_ref, hs2_ref):
    dinv = dinv_ref[...]
    t = (acc_ref[0] + acc_ref[1] + hs1_ref[...]) * dinv + b1_ref[...]
    xemb_ref[...] = t
    h = jnp.maximum(t, 0.0)
    hs2_ref[...] = jnp.dot(h, w2_ref[...], preferred_element_type=_f32) * dinv


def _tc_mid(acc1, hs1, dinv, b1, W2):
    return pl.pallas_call(
        _tcc_body,
        grid=(_GRID,),
        in_specs=[
            pl.BlockSpec((2, _BLK, 128), lambda i: (0, i, 0)),
            pl.BlockSpec((_BLK, 128), lambda i: (i, 0)),
            pl.BlockSpec((_BLK, 1), lambda i: (i, 0)),
            pl.BlockSpec((1, 128), lambda i: (0, 0)),
            pl.BlockSpec((128, 64), lambda i: (0, 0)),
        ],
        out_specs=[
            pl.BlockSpec((_BLK, 128), lambda i: (i, 0)),
            pl.BlockSpec((_BLK, 64), lambda i: (i, 0)),
        ],
        out_shape=[
            jax.ShapeDtypeStruct((N, 128), _f32),
            jax.ShapeDtypeStruct((N, 64), _f32),
        ],
    )(acc1, hs1, dinv, b1, W2)


def _tcd_body(acc_ref, hs2_ref, dinv_ref, b2_ref, o_ref):
    o_ref[...] = ((acc_ref[0] + acc_ref[1] + hs2_ref[...]) * dinv_ref[...]
                  + b2_ref[...])


def _tc_out(acc2, hs2, dinv, b2):
    return pl.pallas_call(
        _tcd_body,
        grid=(_GRID,),
        in_specs=[
            pl.BlockSpec((2, _BLK, 64), lambda i: (0, i, 0)),
            pl.BlockSpec((_BLK, 64), lambda i: (i, 0)),
            pl.BlockSpec((_BLK, 1), lambda i: (i, 0)),
            pl.BlockSpec((1, 64), lambda i: (0, 0)),
        ],
        out_specs=pl.BlockSpec((_BLK, 64), lambda i: (i, 0)),
        out_shape=jax.ShapeDtypeStruct((N, 64), _f32),
    )(acc2, hs2, dinv, b2)


def kernel(x, edge_index, edge_attr, W1, b1, W2, b2):
    pad = EP - E
    src = jnp.concatenate([edge_index[0], jnp.zeros((pad,), jnp.int32)])
    dst = jnp.concatenate([edge_index[1], jnp.zeros((pad,), jnp.int32)])
    ew = jnp.concatenate([edge_attr, jnp.zeros((pad,), _f32)])

    zero8 = jnp.zeros((NP_, 8), _f32)

    src2 = src.reshape(EP // K, K)
    dst2 = dst.reshape(EP // K, K)

    degp = _deg_pass(ew, dst2, zero8).reshape(NC, NP_, 8)
    hs1, dinv = _tc_scale_matmul(x, W1, degp)
    acc1 = _msg128(hs1, src2, dst2, ew).reshape(NC, NP_, 128)
    x_emb, hs2 = _tc_mid(acc1, hs1, dinv, b1.reshape(1, 128), W2)
    acc2 = _msg64(hs2, src2, dst2, ew).reshape(NC, NP_, 64)
    out = _tc_out(acc2, hs2, dinv, b2.reshape(1, 64))
    return (out, x_emb)


# speedup vs baseline: 10.2403x; 1.1881x over previous
import functools

import jax
import jax.numpy as jnp
from jax import lax
from jax.experimental import pallas as pl
from jax.experimental.pallas import tpu as pltpu
from jax.experimental.pallas import tpu_sc as plsc

N = 10000
NP_ = 10240
E = 320000
NC = 2
NS = 16
NW = NC * NS
K = 128
CPW = 80
EP = NW * CPW * K
RPS = NP_ // NS

_f32 = jnp.float32


def _mesh():
    return plsc.VectorSubcoreMesh(core_axis_name="c", subcore_axis_name="s")


DEG_SC = 8


def _deg_body(ew_h, dst2_h, zero_h, out_h, dstv, ewv, rows, sem_s, acc_sh):
    c = lax.axis_index("c")
    s = lax.axis_index("s")

    @pl.when(s == 0)
    def _():
        pltpu.sync_copy(zero_h, acc_sh)
    pltpu.sync_copy(zero_h.at[pl.ds(0, DEG_SC * K)], rows)
    plsc.subcore_barrier()

    w = s * NC + c
    base_r = w * CPW
    col0 = jnp.zeros((16,), jnp.int32)
    lane = lax.iota(jnp.int32, 16)

    def chunk(g, carry):
        offr = base_r + g * DEG_SC
        pltpu.sync_copy(dst2_h.at[pl.ds(offr, DEG_SC)], dstv)
        pltpu.sync_copy(ew_h.at[pl.ds(offr * K, DEG_SC * K)], ewv)

        def sg(t, carry2):
            e0 = t * 16
            plsc.store_scatter(rows, [e0 + lane, col0], ewv[pl.ds(e0, 16)])
            return carry2

        lax.fori_loop(0, (DEG_SC * K) // 16, sg, 0)
        descs = [pltpu.async_copy(rows.at[pl.ds(q * K, K)],
                                  acc_sh.at[dstv.at[q]], sem_s, add=True)
                 for q in range(DEG_SC)]
        for d_ in descs:
            d_.wait()
        return carry

    lax.fori_loop(0, CPW // DEG_SC, chunk, 0)
    plsc.subcore_barrier()

    @pl.when(s == 0)
    def _():
        pltpu.sync_copy(acc_sh, out_h.at[pl.ds(c * NP_, NP_)])


_deg_pass = pl.kernel(
    _deg_body,
    out_type=jax.ShapeDtypeStruct((NC * NP_, 8), _f32),
    mesh=_mesh(),
    compiler_params=pltpu.CompilerParams(needs_layout_passes=False, use_tc_tiling_on_sc=False),
    scratch_types=[
        pltpu.VMEM((DEG_SC, K), jnp.int32),
        pltpu.VMEM((DEG_SC * K,), _f32),
        pltpu.VMEM((DEG_SC * K, 8), _f32),
        pltpu.SemaphoreType.DMA,
        pltpu.VMEM_SHARED((NP_, 8), _f32),
    ],
)


def _msg_body(D, SPS, table_h, src2_h, dst2_h, ew_h, out_h,
              srcv, dstv, ewv, rows, sem_i0, sem_i1, sem_g, sem_s0, sem_s1,
              acc_sh):
    c = lax.axis_index("c")
    s = lax.axis_index("s")
    row0 = s * RPS
    nv = D // 16
    SCK = SPS * K
    G = CPW // SPS
    z16 = jnp.zeros((16,), _f32)

    def zr(r, carry):
        for j in range(nv):
            rows[0, r, pl.ds(j * 16, 16)] = z16
        return carry

    lax.fori_loop(0, SCK, zr, 0)
    nfull, rem = RPS // SCK, RPS % SCK
    for t in range(nfull):
        pltpu.sync_copy(rows.at[0], acc_sh.at[pl.ds(row0 + t * SCK, SCK)])
    if rem:
        pltpu.sync_copy(rows.at[0, pl.ds(0, rem)],
                        acc_sh.at[pl.ds(row0 + nfull * SCK, rem)])
    plsc.subcore_barrier()

    w = s * NC + c
    base_r = w * CPW
    sem_i = (sem_i0, sem_i1)
    sem_s = (sem_s0, sem_s1)

    def idx_issue(g, b):
        offr = base_r + g * SPS
        pltpu.async_copy(src2_h.at[pl.ds(offr, SPS)], srcv.at[b], sem_i[b])
        pltpu.async_copy(dst2_h.at[pl.ds(offr, SPS)], dstv.at[b], sem_i[b])
        pltpu.async_copy(ew_h.at[pl.ds(offr * K, SCK)], ewv.at[b], sem_i[b])

    def idx_wait(b):
        pltpu.make_async_copy(src2_h.at[pl.ds(0, SPS)], srcv.at[b],
                              sem_i[b]).wait()
        pltpu.make_async_copy(dst2_h.at[pl.ds(0, SPS)], dstv.at[b],
                              sem_i[b]).wait()
        pltpu.make_async_copy(ew_h.at[pl.ds(0, SCK)], ewv.at[b],
                              sem_i[b]).wait()

    def gather(b):
        ds_ = [pltpu.async_copy(table_h.at[srcv.at[b, q]],
                                rows.at[b, pl.ds(q * K, K)], sem_g)
               for q in range(SPS)]
        for d_ in ds_:
            d_.wait()

    def scale(b):
        def grp(t, carry):
            e0 = t * 16
            for l in range(16):
                e = e0 + l
                bc = plsc.load_gather(
                    ewv, [jnp.full((16,), b, jnp.int32),
                          jnp.full((16,), e, jnp.int32)])
                for j in range(nv):
                    rows[b, e, pl.ds(j * 16, 16)] = (
                        rows[b, e, pl.ds(j * 16, 16)] * bc)
            return carry

        lax.fori_loop(0, SCK // 16, grp, 0)

    def scat_issue(b):
        for q in range(SPS):
            pltpu.async_copy(rows.at[b, pl.ds(q * K, K)],
                             acc_sh.at[dstv.at[b, q]], sem_s[b], add=True)

    def scat_wait(b):
        for q in range(SPS):
            pltpu.make_async_copy(rows.at[b, pl.ds(q * K, K)],
                                  acc_sh.at[dstv.at[b, q]], sem_s[b]).wait()

    idx_issue(0, 0)
    idx_issue(1, 1)

    def body2(h, carry):
        for b in (0, 1):
            g = 2 * h + b

            @pl.when(g >= 2)
            def _():
                scat_wait(b)

            idx_wait(b)
            gather(b)

            @pl.when(g + 2 < G)
            def _():
                idx_issue(g + 2, b)

            scale(b)
            scat_issue(b)
        return carry

    lax.fori_loop(0, G // 2, body2, 0)
    scat_wait(0)
    scat_wait(1)
    plsc.subcore_barrier()
    pltpu.sync_copy(acc_sh.at[pl.ds(row0, RPS)],
                    out_h.at[pl.ds(c * NP_ + row0, RPS)])


def _make_msg_pass(D, SPS):
    return pl.kernel(
        functools.partial(_msg_body, D, SPS),
        out_type=jax.ShapeDtypeStruct((NC * NP_, D), _f32),
        mesh=_mesh(),
        compiler_params=pltpu.CompilerParams(needs_layout_passes=False, use_tc_tiling_on_sc=False),
        scratch_types=[
            pltpu.VMEM((2, SPS, K), jnp.int32),
            pltpu.VMEM((2, SPS, K), jnp.int32),
            pltpu.VMEM((2, SPS * K), _f32),
            pltpu.VMEM((2, SPS * K, D), _f32),
            pltpu.SemaphoreType.DMA,
            pltpu.SemaphoreType.DMA,
            pltpu.SemaphoreType.DMA,
            pltpu.SemaphoreType.DMA,
            pltpu.SemaphoreType.DMA,
            pltpu.VMEM_SHARED((NP_, D), _f32),
        ],
    )


_msg128 = _make_msg_pass(128, 1)
_msg64 = _make_msg_pass(64, 2)


_BLK = 1000
_GRID = N // _BLK


def _tcb_body(x_ref, w_ref, degp_ref, hs_ref, dinv_ref):
    deg = degp_ref[0, :, 0:1] + degp_ref[1, :, 0:1] + 1.0
    dinv = lax.rsqrt(deg)
    dinv_ref[...] = dinv
    hs_ref[...] = jnp.dot(x_ref[...], w_ref[...],
                          preferred_element_type=_f32) * dinv


def _tc_scale_matmul(x, W1, degp):
    return pl.pallas_call(
        _tcb_body,
        grid=(_GRID,),
        in_specs=[
            pl.BlockSpec((_BLK, 128), lambda i: (i, 0)),
            pl.BlockSpec((128, 128), lambda i: (0, 0)),
            pl.BlockSpec((2, _BLK, 8), lambda i: (0, i, 0)),
        ],
        out_specs=[
            pl.BlockSpec((_BLK, 128), lambda i: (i, 0)),
            pl.BlockSpec((_BLK, 1), lambda i: (i, 0)),
        ],
        out_shape=[
            jax.ShapeDtypeStruct((N, 128), _f32),
            jax.ShapeDtypeStruct((N, 1), _f32),
        ],
    )(x, W1, degp)


def _tcc_body(acc_ref, hs1_ref, dinv_ref, b1_ref, w2_ref, xemb_ref, hs2_ref):
    dinv = dinv_ref[...]
    t = (acc_ref[0] + acc_ref[1] + hs1_ref[...]) * dinv + b1_ref[...]
    xemb_ref[...] = t
    h = jnp.maximum(t, 0.0)
    hs2_ref[...] = jnp.dot(h, w2_ref[...], preferred_element_type=_f32) * dinv


def _tc_mid(acc1, hs1, dinv, b1, W2):
    return pl.pallas_call(
        _tcc_body,
        grid=(_GRID,),
        in_specs=[
            pl.BlockSpec((2, _BLK, 128), lambda i: (0, i, 0)),
            pl.BlockSpec((_BLK, 128), lambda i: (i, 0)),
            pl.BlockSpec((_BLK, 1), lambda i: (i, 0)),
            pl.BlockSpec((1, 128), lambda i: (0, 0)),
            pl.BlockSpec((128, 64), lambda i: (0, 0)),
        ],
        out_specs=[
            pl.BlockSpec((_BLK, 128), lambda i: (i, 0)),
            pl.BlockSpec((_BLK, 64), lambda i: (i, 0)),
        ],
        out_shape=[
            jax.ShapeDtypeStruct((N, 128), _f32),
            jax.ShapeDtypeStruct((N, 64), _f32),
        ],
    )(acc1, hs1, dinv, b1, W2)


def _tcd_body(acc_ref, hs2_ref, dinv_ref, b2_ref, o_ref):
    o_ref[...] = ((acc_ref[0] + acc_ref[1] + hs2_ref[...]) * dinv_ref[...]
                  + b2_ref[...])


def _tc_out(acc2, hs2, dinv, b2):
    return pl.pallas_call(
        _tcd_body,
        grid=(_GRID,),
        in_specs=[
            pl.BlockSpec((2, _BLK, 64), lambda i: (0, i, 0)),
            pl.BlockSpec((_BLK, 64), lambda i: (i, 0)),
            pl.BlockSpec((_BLK, 1), lambda i: (i, 0)),
            pl.BlockSpec((1, 64), lambda i: (0, 0)),
        ],
        out_specs=pl.BlockSpec((_BLK, 64), lambda i: (i, 0)),
        out_shape=jax.ShapeDtypeStruct((N, 64), _f32),
    )(acc2, hs2, dinv, b2)


def kernel(x, edge_index, edge_attr, W1, b1, W2, b2):
    pad = EP - E
    src = jnp.concatenate([edge_index[0], jnp.zeros((pad,), jnp.int32)])
    dst = jnp.concatenate([edge_index[1], jnp.zeros((pad,), jnp.int32)])
    ew = jnp.concatenate([edge_attr, jnp.zeros((pad,), _f32)])

    zero8 = jnp.zeros((NP_, 8), _f32)

    src2 = src.reshape(EP // K, K)
    dst2 = dst.reshape(EP // K, K)

    degp = _deg_pass(ew, dst2, zero8).reshape(NC, NP_, 8)
    hs1, dinv = _tc_scale_matmul(x, W1, degp)
    acc1 = _msg128(hs1, src2, dst2, ew).reshape(NC, NP_, 128)
    x_emb, hs2 = _tc_mid(acc1, hs1, dinv, b1.reshape(1, 128), W2)
    acc2 = _msg64(hs2, src2, dst2, ew).reshape(NC, NP_, 64)
    out = _tc_out(acc2, hs2, dinv, b2.reshape(1, 64))
    return (out, x_emb)
